# Initial kernel scaffold; baseline (speedup 1.0000x reference)
import jax
import jax.numpy as jnp
from jax.experimental import pallas as pl


def kernel(x, training, start_w, start_b, w1, b1, w2, b2, w_gate):
    raise NotImplementedError("write your pallas kernel here")


# speedup vs baseline: 1.1920x; 1.1920x over previous
import numpy as np
import jax
import jax.numpy as jnp
from jax.experimental import pallas as pl
from jax.experimental.pallas import tpu as pltpu

_B = 32
_L = 4096
_F = 64
_K = 2048
_H = 8192
_P = 126
_PPAD = 128

_l_idx = np.arange(_L, dtype=np.int64)[:, None]
_k_idx = np.arange(1, _K + 1, dtype=np.int64)[None, :]
_ang = (2.0 * np.pi / _L) * ((_l_idx * _k_idx) % _L).astype(np.float64)
_SCALE = 1.0 / np.sqrt(_L)
_DFT_C = np.ascontiguousarray((np.cos(_ang) * _SCALE).astype(np.float32))
_DFT_S = np.ascontiguousarray((-np.sin(_ang) * _SCALE).astype(np.float32))
del _l_idx, _k_idx, _ang


def _startfc_body(x_ref, w_ref, b_ref, o_ref):
    xb = x_ref[...]
    r = jax.lax.dot_general(xb, w_ref[...],
                            (((2,), (0,)), ((), ())),
                            preferred_element_type=jnp.float32)
    o_ref[...] = r[..., 0] + b_ref[...]


def _dft_body(xp_ref, c_ref, s_ref, o_ref):
    xp = xp_ref[...]
    o_ref[0:_B, :] = jnp.dot(xp, c_ref[...], preferred_element_type=jnp.float32)
    o_ref[_B:2 * _B, :] = jnp.dot(xp, s_ref[...], preferred_element_type=jnp.float32)


def _mlp_body(xs_ref, w1_ref, b1_ref, w2_ref, b2_ref, wg_ref, o_ref,
              q0_ref, q1_ref):
    i = pl.program_id(0)

    @pl.when(i == 0)
    def _init():
        q0_ref[...] = jnp.zeros_like(q0_ref)
        q1_ref[...] = jnp.zeros_like(q1_ref)

    xs = xs_ref[...]
    p0 = jnp.dot(xs, w1_ref[0], preferred_element_type=jnp.float32)
    p1 = jnp.dot(xs, w1_ref[1], preferred_element_type=jnp.float32)
    o1r = jnp.maximum(p0[0:_B] - p1[_B:2 * _B] + b1_ref[0:1, :], 0.0)
    o1i = jnp.maximum(p0[_B:2 * _B] + p1[0:_B] + b1_ref[1:2, :], 0.0)
    o1 = jnp.concatenate([o1r, o1i], axis=0)
    q0_ref[...] += jnp.dot(o1, w2_ref[0], preferred_element_type=jnp.float32)
    q1_ref[...] += jnp.dot(o1, w2_ref[1], preferred_element_type=jnp.float32)

    @pl.when(i == pl.num_programs(0) - 1)
    def _fini():
        q0 = q0_ref[...]
        q1 = q1_ref[...]
        o2r = q0[0:_B] - q1[_B:2 * _B] + b2_ref[0:1, :]
        o2i = q0[_B:2 * _B] + q1[0:_B] + b2_ref[1:2, :]
        amp = jnp.sqrt(o2r * o2r + o2i * o2i)
        o_ref[...] = jnp.dot(amp, wg_ref[...], preferred_element_type=jnp.float32)


def _gate_body(l_ref, o_ref):
    lg = l_ref[...]
    col = jax.lax.broadcasted_iota(jnp.int32, (_B, _PPAD), 1)
    neg = jnp.float32(-3e38)
    big = jnp.int32(1 << 30)
    lm = jnp.where(col < _P, lg, neg)
    m1 = jnp.max(lm, axis=1, keepdims=True)
    i1 = jnp.min(jnp.where(lm == m1, col, big), axis=1, keepdims=True)
    lm2 = jnp.where(col == i1, neg, lm)
    m2 = jnp.max(lm2, axis=1, keepdims=True)
    i2 = jnp.min(jnp.where(lm2 == m2, col, big), axis=1, keepdims=True)
    e = jnp.exp(m2 - m1)
    w1v = 1.0 / (1.0 + e)
    w2v = e / (1.0 + e)
    o_ref[...] = (jnp.where(col == i1, w1v, 0.0)
                  + jnp.where(col == i2, w2v, 0.0))


_BBLK = 8
_HBLK = 512
_KBLK = 512


def kernel(x, training, start_w, start_b, w1, b1, w2, b2, w_gate):
    del training
    f32 = jnp.float32
    dft_c = jnp.asarray(_DFT_C)
    dft_s = jnp.asarray(_DFT_S)
    start_b2 = jnp.reshape(start_b, (1, 1)).astype(f32)
    wg_pad = jnp.pad(w_gate, ((0, 0), (0, _PPAD - _P)))

    xp = pl.pallas_call(
        _startfc_body,
        grid=(_B // _BBLK,),
        in_specs=[
            pl.BlockSpec((_BBLK, _L, _F), lambda i: (i, 0, 0)),
            pl.BlockSpec((_F, 1), lambda i: (0, 0)),
            pl.BlockSpec((1, 1), lambda i: (0, 0)),
        ],
        out_specs=pl.BlockSpec((_BBLK, _L), lambda i: (i, 0)),
        out_shape=jax.ShapeDtypeStruct((_B, _L), f32),
    )(x, start_w, start_b2)

    xs = pl.pallas_call(
        _dft_body,
        grid=(_K // _KBLK,),
        in_specs=[
            pl.BlockSpec((_B, _L), lambda i: (0, 0)),
            pl.BlockSpec((_L, _KBLK), lambda i: (0, i)),
            pl.BlockSpec((_L, _KBLK), lambda i: (0, i)),
        ],
        out_specs=pl.BlockSpec((2 * _B, _KBLK), lambda i: (0, i)),
        out_shape=jax.ShapeDtypeStruct((2 * _B, _K), f32),
    )(xp, dft_c, dft_s)

    logits = pl.pallas_call(
        _mlp_body,
        grid=(_H // _HBLK,),
        in_specs=[
            pl.BlockSpec((2 * _B, _K), lambda i: (0, 0)),
            pl.BlockSpec((2, _K, _HBLK), lambda i: (0, 0, i)),
            pl.BlockSpec((2, _HBLK), lambda i: (0, i)),
            pl.BlockSpec((2, _HBLK, _K), lambda i: (0, i, 0)),
            pl.BlockSpec((2, _K), lambda i: (0, 0)),
            pl.BlockSpec((_K, _PPAD), lambda i: (0, 0)),
        ],
        out_specs=pl.BlockSpec((_B, _PPAD), lambda i: (0, 0)),
        out_shape=jax.ShapeDtypeStruct((_B, _PPAD), f32),
        scratch_shapes=[
            pltpu.VMEM((2 * _B, _K), f32),
            pltpu.VMEM((2 * _B, _K), f32),
        ],
        compiler_params=pltpu.CompilerParams(
            dimension_semantics=("arbitrary",)),
    )(xs, w1, b1, w2, b2, wg_pad)

    gates = pl.pallas_call(
        _gate_body,
        in_specs=[pl.BlockSpec((_B, _PPAD), lambda: (0, 0))],
        out_specs=pl.BlockSpec((_B, _PPAD), lambda: (0, 0)),
        out_shape=jax.ShapeDtypeStruct((_B, _PPAD), f32),
    )(logits)

    return gates[:, :_P]
